# baseline (speedup 1.0000x reference)
import jax
import jax.numpy as jnp
from jax import lax
from jax.experimental import pallas as pl
from jax.experimental.pallas import tpu as pltpu
from jax.experimental.pallas import tpu_sc as plsc

_IDX_B = (1, 3, 5, 7, 9, 11)
_IDX_I = (2, 4, 6, 8, 10, 12)
_L0 = 512
_L = 510
_C = 6
_K = 4
_B = 8
_NL = 13
_PAD = -3e8
_EXCL = -3.5e8
_IBIG = (1 << 30)
_FBIG = 3e8


def _sc_gather(rowidx_hbm, table_hbm, out_hbm, idx_s, sems):
    cid = lax.axis_index("c")

    @pl.when(cid == 0)
    def _():
        pltpu.sync_copy(rowidx_hbm, idx_s)
        copies = []
        for e in range(_B):
            copies.append(pltpu.async_copy(
                table_hbm.at[e, idx_s[e]], out_hbm.at[e], sems.at[e]))
        for c in copies:
            c.wait()


def _shl(a, s, fill):
    pad = jnp.full((a.shape[0], s), jnp.float32(fill))
    return jnp.concatenate([a[:, s:], pad], axis=1)


def _body(v_ref, vl_ref, lp_ref, out_ref):
    ex = pl.program_id(0)
    g = lp_ref[0]
    gt = jnp.transpose(g, (1, 0))

    cb = jnp.concatenate([gt[b:b + 1, :] for b in _IDX_B], axis=0)
    ci = jnp.concatenate([gt[x:x + 1, :] for x in _IDX_I], axis=0)
    neg = jnp.log(jnp.maximum(1.0 - jnp.exp(ci), 1e-06))

    jj = jax.lax.broadcasted_iota(jnp.int32, (_C, _L0), 1)
    valid = jj < _L

    xl = jnp.where(valid,
                   jnp.minimum(_shl(ci, 1, 0.0), _shl(neg, 2, 0.0)),
                   jnp.float32(_PAD))
    sm = xl
    s = 1
    while s < _L0:
        sm = jnp.maximum(sm, _shl(sm, s, _PAD))
        s *= 2
    rmv = jnp.where(valid, jnp.minimum(cb, sm), jnp.float32(_PAD))

    byl = jnp.maximum(cb, ci)
    c1l = jnp.log(jnp.maximum(
        1.0 - jnp.exp(jnp.minimum(_shl(cb, 1, 0.0), _shl(ci, 2, 0.0))),
        1e-06))
    nnl = jnp.maximum(_shl(neg, 1, 0.0), _shl(neg, 2, 0.0))

    vals_ks = []
    rhs_ks = []
    im_hist = []
    jm_hist = []
    for k in range(_K):
        m6 = jnp.max(rmv, axis=1, keepdims=True)
        im6 = jnp.min(jnp.where(rmv == m6, jj, jnp.int32(_IBIG)),
                      axis=1, keepdims=True)
        bsel = jnp.min(jnp.where(jj == im6, cb, jnp.float32(_FBIG)),
                       axis=1, keepdims=True)
        row = jnp.minimum(bsel, xl) + jnp.where(
            jj < im6, jnp.float32(-1e8), jnp.float32(0.0))
        for t in range(k):
            row = jnp.where((im6 == im_hist[t]) & (jj == jm_hist[t]),
                            jnp.float32(_EXCL), row)
        jm6 = jnp.min(jnp.where(row == m6, jj, jnp.int32(_IBIG)),
                      axis=1, keepdims=True)
        rowx = jnp.where(jj == jm6, jnp.float32(_EXCL), row)
        rmv = jnp.where(jj == im6,
                        jnp.max(rowx, axis=1, keepdims=True), rmv)
        by_s = jnp.min(jnp.where(jj == im6, byl, jnp.float32(_FBIG)),
                       axis=1, keepdims=True)
        c1_s = jnp.min(jnp.where(jj == jm6, c1l, jnp.float32(_FBIG)),
                       axis=1, keepdims=True)
        nn_s = jnp.min(jnp.where(jj == jm6, nnl, jnp.float32(_FBIG)),
                       axis=1, keepdims=True)
        vals_ks.append(m6)
        rhs_ks.append(jnp.minimum(c1_s, jnp.maximum(by_s, nn_s)))
        im_hist.append(im6)
        jm_hist.append(jm6)

    ii6 = jax.lax.broadcasted_iota(jnp.int32, (_C, 1), 0)
    loss = jnp.zeros((1, 1), jnp.float32)
    for k in range(_K):
        r6 = rhs_ks[k]
        m1 = jnp.min(r6, axis=0, keepdims=True)
        am = jnp.min(jnp.where(r6 == m1, ii6, jnp.int32(_IBIG)),
                     axis=0, keepdims=True)
        m2 = jnp.min(jnp.where(ii6 == am, jnp.float32(_FBIG), r6),
                     axis=0, keepdims=True)
        mex = jnp.where(ii6 == am, m2, m1)
        loss = loss + jnp.sum(jnp.maximum(vals_ks[k] - mex, 0.0),
                              axis=0, keepdims=True)

    loss = jnp.where(vl_ref[ex] > 0, loss, jnp.zeros((1, 1), jnp.float32))

    @pl.when(ex == 0)
    def _init():
        out_ref[...] = jnp.zeros((1, 1), jnp.float32)

    out_ref[...] = out_ref[...] + loss

    @pl.when(ex == _B - 1)
    def _fin():
        num_prop = vl_ref[0]
        for t in range(1, _B):
            num_prop = num_prop + vl_ref[t]
        out_ref[...] = out_ref[...] / jnp.maximum(
            num_prop, 1).astype(jnp.float32)


def kernel(log_pa, score, v_label, v_l, role_label, roleset_id, extra):
    b = log_pa.shape[0]
    v_idx = v_label[:, 0].astype(jnp.int32)

    gather = pl.kernel(
        _sc_gather,
        mesh=plsc.ScalarSubcoreMesh(axis_name="c", num_cores=2),
        out_type=jax.ShapeDtypeStruct((b, _L0, _NL), jnp.float32),
        scratch_types=[
            pltpu.SMEM((b,), jnp.int32),
            pltpu.SemaphoreType.DMA((b,)),
        ],
    )
    gathered = gather(v_idx, log_pa)

    out = pl.pallas_call(
        _body,
        grid_spec=pltpu.PrefetchScalarGridSpec(
            num_scalar_prefetch=2,
            grid=(b,),
            in_specs=[
                pl.BlockSpec((1, _L0, _NL), lambda i, v, vl: (i, 0, 0)),
            ],
            out_specs=pl.BlockSpec((1, 1), lambda i, v, vl: (0, 0)),
        ),
        out_shape=jax.ShapeDtypeStruct((1, 1), jnp.float32),
    )(v_idx, v_l.astype(jnp.int32), gathered)
    return out.reshape(1)


# speedup vs baseline: 2.9399x; 2.9399x over previous
import jax
import jax.numpy as jnp
from jax import lax
from jax.experimental import pallas as pl
from jax.experimental.pallas import tpu as pltpu
from jax.experimental.pallas import tpu_sc as plsc

_IDX_B = (1, 3, 5, 7, 9, 11)
_IDX_I = (2, 4, 6, 8, 10, 12)
_L0 = 512
_L = 510
_C = 6
_K = 4
_B = 8
_NL = 13
_PAD = -3e8
_EXCL = -3.5e8
_IBIG = (1 << 30)
_FBIG = 3e8


def _sc_gather(rowidx_hbm, table_hbm, out_hbm, idx_s, sems):
    cid = lax.axis_index("c")
    half = _B // 2

    @pl.when(cid < 2)
    def _():
        pltpu.sync_copy(rowidx_hbm, idx_s)
        lo = cid * half
        copies = []
        for e in range(half):
            copies.append(pltpu.async_copy(
                table_hbm.at[idx_s[lo + e]],
                out_hbm.at[lo + e], sems.at[lo + e]))
        for c in copies:
            c.wait()


def _shl(a, s, fill):
    pad = jnp.full((a.shape[0], s), jnp.float32(fill))
    return jnp.concatenate([a[:, s:], pad], axis=1)


def _body(v_ref, vl_ref, lp_ref, out_ref):
    ex = pl.program_id(0)
    g = lp_ref[0]
    gt = jnp.transpose(g, (1, 0))

    cb = jnp.concatenate([gt[b:b + 1, :] for b in _IDX_B], axis=0)
    ci = jnp.concatenate([gt[x:x + 1, :] for x in _IDX_I], axis=0)
    neg = jnp.log(jnp.maximum(1.0 - jnp.exp(ci), 1e-06))

    jj = jax.lax.broadcasted_iota(jnp.int32, (_C, _L0), 1)
    valid = jj < _L

    xl = jnp.where(valid,
                   jnp.minimum(_shl(ci, 1, 0.0), _shl(neg, 2, 0.0)),
                   jnp.float32(_PAD))
    sm = xl
    s = 1
    while s < _L0:
        sm = jnp.maximum(sm, _shl(sm, s, _PAD))
        s *= 2
    rmv = jnp.where(valid, jnp.minimum(cb, sm), jnp.float32(_PAD))

    byl = jnp.maximum(cb, ci)
    c1l = jnp.log(jnp.maximum(
        1.0 - jnp.exp(jnp.minimum(_shl(cb, 1, 0.0), _shl(ci, 2, 0.0))),
        1e-06))
    nnl = jnp.maximum(_shl(neg, 1, 0.0), _shl(neg, 2, 0.0))

    vals_ks = []
    rhs_ks = []
    im_hist = []
    jm_hist = []
    for k in range(_K):
        m6 = jnp.max(rmv, axis=1, keepdims=True)
        im6 = jnp.min(jnp.where(rmv == m6, jj, jnp.int32(_IBIG)),
                      axis=1, keepdims=True)
        bsel = jnp.min(jnp.where(jj == im6, cb, jnp.float32(_FBIG)),
                       axis=1, keepdims=True)
        row = jnp.minimum(bsel, xl) + jnp.where(
            jj < im6, jnp.float32(-1e8), jnp.float32(0.0))
        for t in range(k):
            row = jnp.where((im6 == im_hist[t]) & (jj == jm_hist[t]),
                            jnp.float32(_EXCL), row)
        jm6 = jnp.min(jnp.where(row == m6, jj, jnp.int32(_IBIG)),
                      axis=1, keepdims=True)
        rowx = jnp.where(jj == jm6, jnp.float32(_EXCL), row)
        rmv = jnp.where(jj == im6,
                        jnp.max(rowx, axis=1, keepdims=True), rmv)
        by_s = jnp.min(jnp.where(jj == im6, byl, jnp.float32(_FBIG)),
                       axis=1, keepdims=True)
        c1_s = jnp.min(jnp.where(jj == jm6, c1l, jnp.float32(_FBIG)),
                       axis=1, keepdims=True)
        nn_s = jnp.min(jnp.where(jj == jm6, nnl, jnp.float32(_FBIG)),
                       axis=1, keepdims=True)
        vals_ks.append(m6)
        rhs_ks.append(jnp.minimum(c1_s, jnp.maximum(by_s, nn_s)))
        im_hist.append(im6)
        jm_hist.append(jm6)

    ii6 = jax.lax.broadcasted_iota(jnp.int32, (_C, 1), 0)
    loss = jnp.zeros((1, 1), jnp.float32)
    for k in range(_K):
        r6 = rhs_ks[k]
        m1 = jnp.min(r6, axis=0, keepdims=True)
        am = jnp.min(jnp.where(r6 == m1, ii6, jnp.int32(_IBIG)),
                     axis=0, keepdims=True)
        m2 = jnp.min(jnp.where(ii6 == am, jnp.float32(_FBIG), r6),
                     axis=0, keepdims=True)
        mex = jnp.where(ii6 == am, m2, m1)
        loss = loss + jnp.sum(jnp.maximum(vals_ks[k] - mex, 0.0),
                              axis=0, keepdims=True)

    loss = jnp.where(vl_ref[ex] > 0, loss, jnp.zeros((1, 1), jnp.float32))

    @pl.when(ex == 0)
    def _init():
        out_ref[...] = jnp.zeros((1, 1), jnp.float32)

    out_ref[...] = out_ref[...] + loss

    @pl.when(ex == _B - 1)
    def _fin():
        num_prop = vl_ref[0]
        for t in range(1, _B):
            num_prop = num_prop + vl_ref[t]
        out_ref[...] = out_ref[...] / jnp.maximum(
            num_prop, 1).astype(jnp.float32)


def kernel(log_pa, score, v_label, v_l, role_label, roleset_id, extra):
    b = log_pa.shape[0]
    v_idx = v_label[:, 0].astype(jnp.int32)
    row_idx = jnp.arange(b, dtype=jnp.int32) * _L0 + v_idx
    table = log_pa.reshape(b * _L0, _L0, _NL)

    gather = pl.kernel(
        _sc_gather,
        mesh=plsc.ScalarSubcoreMesh(axis_name="c", num_cores=2),
        out_type=jax.ShapeDtypeStruct((b, _L0, _NL), jnp.float32),
        scratch_types=[
            pltpu.SMEM((b,), jnp.int32),
            pltpu.SemaphoreType.DMA((b,)),
        ],
    )
    gathered = gather(row_idx, table)

    out = pl.pallas_call(
        _body,
        grid_spec=pltpu.PrefetchScalarGridSpec(
            num_scalar_prefetch=2,
            grid=(b,),
            in_specs=[
                pl.BlockSpec((1, _L0, _NL), lambda i, v, vl: (i, 0, 0)),
            ],
            out_specs=pl.BlockSpec((1, 1), lambda i, v, vl: (0, 0)),
        ),
        out_shape=jax.ShapeDtypeStruct((1, 1), jnp.float32),
    )(v_idx, v_l.astype(jnp.int32), gathered)
    return out.reshape(1)


# speedup vs baseline: 3.2137x; 1.0931x over previous
import jax
import jax.numpy as jnp
from jax import lax
from jax.experimental import pallas as pl
from jax.experimental.pallas import tpu as pltpu
from jax.experimental.pallas import tpu_sc as plsc

_IDX_B = (1, 3, 5, 7, 9, 11)
_IDX_I = (2, 4, 6, 8, 10, 12)
_L0 = 512
_L = 510
_C = 6
_K = 4
_B = 8
_NL = 13
_PAD = -3e8
_EXCL = -3.5e8
_IBIG = (1 << 30)
_FBIG = 3e8


def _sc_gather(rowidx_hbm, table_hbm, out_hbm, idx_s, sems):
    cid = lax.axis_index("c")
    half = _B // 2

    @pl.when(cid < 2)
    def _():
        pltpu.sync_copy(rowidx_hbm, idx_s)
        lo = cid * half
        copies = []
        for e in range(half):
            copies.append(pltpu.async_copy(
                table_hbm.at[idx_s[lo + e]],
                out_hbm.at[lo + e], sems.at[lo + e]))
        for c in copies:
            c.wait()


def _shl(a, s, fill):
    pad = jnp.full((a.shape[0], s), jnp.float32(fill))
    return jnp.concatenate([a[:, s:], pad], axis=1)


def _example_loss(gt, jj, valid, ii6):
    cb = jnp.concatenate([gt[b:b + 1, :] for b in _IDX_B], axis=0)
    ci = jnp.concatenate([gt[x:x + 1, :] for x in _IDX_I], axis=0)
    neg = jnp.log(jnp.maximum(1.0 - jnp.exp(ci), 1e-06))

    xl = jnp.where(valid,
                   jnp.minimum(_shl(ci, 1, 0.0), _shl(neg, 2, 0.0)),
                   jnp.float32(_PAD))
    sm = xl
    s = 1
    while s < _L0:
        sm = jnp.maximum(sm, _shl(sm, s, _PAD))
        s *= 2
    rmv = jnp.where(valid, jnp.minimum(cb, sm), jnp.float32(_PAD))

    byl = jnp.maximum(cb, ci)
    c1l = jnp.log(jnp.maximum(
        1.0 - jnp.exp(jnp.minimum(_shl(cb, 1, 0.0), _shl(ci, 2, 0.0))),
        1e-06))
    nnl = jnp.maximum(_shl(neg, 1, 0.0), _shl(neg, 2, 0.0))

    vals_ks = []
    rhs_ks = []
    im_hist = []
    jm_hist = []
    for k in range(_K):
        m6 = jnp.max(rmv, axis=1, keepdims=True)
        im6 = jnp.min(jnp.where(rmv == m6, jj, jnp.int32(_IBIG)),
                      axis=1, keepdims=True)
        bsel = jnp.min(jnp.where(jj == im6, cb, jnp.float32(_FBIG)),
                       axis=1, keepdims=True)
        row = jnp.minimum(bsel, xl) + jnp.where(
            jj < im6, jnp.float32(-1e8), jnp.float32(0.0))
        for t in range(k):
            row = jnp.where((im6 == im_hist[t]) & (jj == jm_hist[t]),
                            jnp.float32(_EXCL), row)
        jm6 = jnp.min(jnp.where(row == m6, jj, jnp.int32(_IBIG)),
                      axis=1, keepdims=True)
        rowx = jnp.where(jj == jm6, jnp.float32(_EXCL), row)
        rmv = jnp.where(jj == im6,
                        jnp.max(rowx, axis=1, keepdims=True), rmv)
        by_s = jnp.min(jnp.where(jj == im6, byl, jnp.float32(_FBIG)),
                       axis=1, keepdims=True)
        c1_s = jnp.min(jnp.where(jj == jm6, c1l, jnp.float32(_FBIG)),
                       axis=1, keepdims=True)
        nn_s = jnp.min(jnp.where(jj == jm6, nnl, jnp.float32(_FBIG)),
                       axis=1, keepdims=True)
        vals_ks.append(m6)
        rhs_ks.append(jnp.minimum(c1_s, jnp.maximum(by_s, nn_s)))
        im_hist.append(im6)
        jm_hist.append(jm6)

    loss = jnp.zeros((1, 1), jnp.float32)
    for k in range(_K):
        r6 = rhs_ks[k]
        m1 = jnp.min(r6, axis=0, keepdims=True)
        am = jnp.min(jnp.where(r6 == m1, ii6, jnp.int32(_IBIG)),
                     axis=0, keepdims=True)
        m2 = jnp.min(jnp.where(ii6 == am, jnp.float32(_FBIG), r6),
                     axis=0, keepdims=True)
        mex = jnp.where(ii6 == am, m2, m1)
        loss = loss + jnp.sum(jnp.maximum(vals_ks[k] - mex, 0.0),
                              axis=0, keepdims=True)
    return loss


def _body(v_ref, vl_ref, lp_ref, out_ref):
    jj = jax.lax.broadcasted_iota(jnp.int32, (_C, _L0), 1)
    valid = jj < _L
    ii6 = jax.lax.broadcasted_iota(jnp.int32, (_C, 1), 0)

    total = jnp.zeros((1, 1), jnp.float32)
    for ex in range(_B):
        gt = jnp.transpose(lp_ref[ex], (1, 0))
        loss = _example_loss(gt, jj, valid, ii6)
        total = total + jnp.where(vl_ref[ex] > 0, loss,
                                  jnp.zeros((1, 1), jnp.float32))

    num_prop = vl_ref[0]
    for t in range(1, _B):
        num_prop = num_prop + vl_ref[t]
    out_ref[...] = total / jnp.maximum(num_prop, 1).astype(jnp.float32)


def kernel(log_pa, score, v_label, v_l, role_label, roleset_id, extra):
    b = log_pa.shape[0]
    v_idx = v_label[:, 0].astype(jnp.int32)
    row_idx = jnp.arange(b, dtype=jnp.int32) * _L0 + v_idx
    table = log_pa.reshape(b * _L0, _L0, _NL)

    gather = pl.kernel(
        _sc_gather,
        mesh=plsc.ScalarSubcoreMesh(axis_name="c", num_cores=2),
        out_type=jax.ShapeDtypeStruct((b, _L0, _NL), jnp.float32),
        scratch_types=[
            pltpu.SMEM((b,), jnp.int32),
            pltpu.SemaphoreType.DMA((b,)),
        ],
    )
    gathered = gather(row_idx, table)

    out = pl.pallas_call(
        _body,
        grid_spec=pltpu.PrefetchScalarGridSpec(
            num_scalar_prefetch=2,
            grid=(1,),
            in_specs=[
                pl.BlockSpec((b, _L0, _NL), lambda i, v, vl: (0, 0, 0)),
            ],
            out_specs=pl.BlockSpec((1, 1), lambda i, v, vl: (0, 0)),
        ),
        out_shape=jax.ShapeDtypeStruct((1, 1), jnp.float32),
    )(v_idx, v_l.astype(jnp.int32), gathered)
    return out.reshape(1)


# speedup vs baseline: 4.1607x; 1.2947x over previous
import jax
import jax.numpy as jnp
from jax import lax
from jax.experimental import pallas as pl
from jax.experimental.pallas import tpu as pltpu
from jax.experimental.pallas import tpu_sc as plsc

_IDX_B = (1, 3, 5, 7, 9, 11)
_IDX_I = (2, 4, 6, 8, 10, 12)
_L0 = 512
_L = 510
_C = 6
_K = 4
_B = 8
_NL = 13
_PAD = -3e8
_EXCL = -3.5e8
_IBIG = (1 << 30)
_FBIG = 3e8


def _sc_gather(rowidx_hbm, table_hbm, out_hbm, idx_s, sems):
    cid = lax.axis_index("c")
    half = _B // 2

    @pl.when(cid < 2)
    def _():
        pltpu.sync_copy(rowidx_hbm, idx_s)
        lo = cid * half
        copies = []
        for e in range(half):
            copies.append(pltpu.async_copy(
                table_hbm.at[idx_s[lo + e]],
                out_hbm.at[lo + e], sems.at[lo + e]))
        for c in copies:
            c.wait()


def _shl(a, s, fill):
    pad = jnp.full((a.shape[0], s), jnp.float32(fill))
    return jnp.concatenate([a[:, s:], pad], axis=1)


def _example_loss(gt, jj, valid, ii6):
    cb = jnp.concatenate([gt[b:b + 1, :] for b in _IDX_B], axis=0)
    ci = jnp.concatenate([gt[x:x + 1, :] for x in _IDX_I], axis=0)
    neg = jnp.log(jnp.maximum(1.0 - jnp.exp(ci), 1e-06))

    xl = jnp.where(valid,
                   jnp.minimum(_shl(ci, 1, 0.0), _shl(neg, 2, 0.0)),
                   jnp.float32(_PAD))
    sm = xl
    s = 1
    while s < _L0:
        sm = jnp.maximum(sm, _shl(sm, s, _PAD))
        s *= 2
    rmv = jnp.where(valid, jnp.minimum(cb, sm), jnp.float32(_PAD))

    byl = jnp.maximum(cb, ci)
    c1l = jnp.log(jnp.maximum(
        1.0 - jnp.exp(jnp.minimum(_shl(cb, 1, 0.0), _shl(ci, 2, 0.0))),
        1e-06))
    nnl = jnp.maximum(_shl(neg, 1, 0.0), _shl(neg, 2, 0.0))

    vals_ks = []
    rhs_ks = []
    im_hist = []
    jm_hist = []
    for k in range(_K):
        m6 = jnp.max(rmv, axis=1, keepdims=True)
        im6 = jnp.min(jnp.where(rmv == m6, jj, jnp.int32(_IBIG)),
                      axis=1, keepdims=True)
        bsel = jnp.min(jnp.where(jj == im6, cb, jnp.float32(_FBIG)),
                       axis=1, keepdims=True)
        row = jnp.minimum(bsel, xl) + jnp.where(
            jj < im6, jnp.float32(-1e8), jnp.float32(0.0))
        for t in range(k):
            row = jnp.where((im6 == im_hist[t]) & (jj == jm_hist[t]),
                            jnp.float32(_EXCL), row)
        jm6 = jnp.min(jnp.where(row == m6, jj, jnp.int32(_IBIG)),
                      axis=1, keepdims=True)
        rowx = jnp.where(jj == jm6, jnp.float32(_EXCL), row)
        rmv = jnp.where(jj == im6,
                        jnp.max(rowx, axis=1, keepdims=True), rmv)
        by_s = jnp.min(jnp.where(jj == im6, byl, jnp.float32(_FBIG)),
                       axis=1, keepdims=True)
        c1_s = jnp.min(jnp.where(jj == jm6, c1l, jnp.float32(_FBIG)),
                       axis=1, keepdims=True)
        nn_s = jnp.min(jnp.where(jj == jm6, nnl, jnp.float32(_FBIG)),
                       axis=1, keepdims=True)
        vals_ks.append(m6)
        rhs_ks.append(jnp.minimum(c1_s, jnp.maximum(by_s, nn_s)))
        im_hist.append(im6)
        jm_hist.append(jm6)

    loss = jnp.zeros((1, 1), jnp.float32)
    for k in range(_K):
        r6 = rhs_ks[k]
        m1 = jnp.min(r6, axis=0, keepdims=True)
        am = jnp.min(jnp.where(r6 == m1, ii6, jnp.int32(_IBIG)),
                     axis=0, keepdims=True)
        m2 = jnp.min(jnp.where(ii6 == am, jnp.float32(_FBIG), r6),
                     axis=0, keepdims=True)
        mex = jnp.where(ii6 == am, m2, m1)
        loss = loss + jnp.sum(jnp.maximum(vals_ks[k] - mex, 0.0),
                              axis=0, keepdims=True)
    return loss


def _body(ridx_ref, vl_ref, lp_ref, out_ref, gbuf_ref, sems_ref):
    copies = []
    for e in range(_B):
        copies.append(pltpu.make_async_copy(
            lp_ref.at[ridx_ref[e]], gbuf_ref.at[e], sems_ref.at[e]))
    for c in copies:
        c.start()

    jj = jax.lax.broadcasted_iota(jnp.int32, (_C, _L0), 1)
    valid = jj < _L
    ii6 = jax.lax.broadcasted_iota(jnp.int32, (_C, 1), 0)

    total = jnp.zeros((1, 1), jnp.float32)
    for ex in range(_B):
        copies[ex].wait()
        gt = jnp.transpose(gbuf_ref[ex], (1, 0))
        loss = _example_loss(gt, jj, valid, ii6)
        total = total + jnp.where(vl_ref[ex] > 0, loss,
                                  jnp.zeros((1, 1), jnp.float32))

    num_prop = vl_ref[0]
    for t in range(1, _B):
        num_prop = num_prop + vl_ref[t]
    out_ref[...] = total / jnp.maximum(num_prop, 1).astype(jnp.float32)


def kernel(log_pa, score, v_label, v_l, role_label, roleset_id, extra):
    b = log_pa.shape[0]
    v_idx = v_label[:, 0].astype(jnp.int32)
    row_idx = jnp.arange(b, dtype=jnp.int32) * _L0 + v_idx
    table = log_pa.reshape(b * _L0, _L0, _NL)

    out = pl.pallas_call(
        _body,
        grid_spec=pltpu.PrefetchScalarGridSpec(
            num_scalar_prefetch=2,
            grid=(1,),
            in_specs=[
                pl.BlockSpec(memory_space=pl.ANY),
            ],
            out_specs=pl.BlockSpec((1, 1), lambda i, v, vl: (0, 0)),
            scratch_shapes=[
                pltpu.VMEM((b, _L0, _NL), jnp.float32),
                pltpu.SemaphoreType.DMA((b,)),
            ],
        ),
        out_shape=jax.ShapeDtypeStruct((1, 1), jnp.float32),
    )(row_idx, v_l.astype(jnp.int32), table)
    return out.reshape(1)


# speedup vs baseline: 4.1649x; 1.0010x over previous
import jax
import jax.numpy as jnp
from jax.experimental import pallas as pl
from jax.experimental.pallas import tpu as pltpu

_IDX_B = (1, 3, 5, 7, 9, 11)
_IDX_I = (2, 4, 6, 8, 10, 12)
_L0 = 512
_L = 510
_C = 6
_K = 4
_B = 8
_NL = 13
_PAD = -3e8
_EXCL = -3.5e8
_IBIG = (1 << 30)
_FBIG = 3e8


def _shl(a, s, fill):
    pad = jnp.full((a.shape[0], s), jnp.float32(fill))
    return jnp.concatenate([a[:, s:], pad], axis=1)


def _example_loss(gt, jj, valid, ii6):
    cb = jnp.concatenate([gt[b:b + 1, :] for b in _IDX_B], axis=0)
    ci = jnp.concatenate([gt[x:x + 1, :] for x in _IDX_I], axis=0)
    neg = jnp.log(jnp.maximum(1.0 - jnp.exp(ci), 1e-06))

    xl = jnp.where(valid,
                   jnp.minimum(_shl(ci, 1, 0.0), _shl(neg, 2, 0.0)),
                   jnp.float32(_PAD))
    sm = xl
    s = 1
    while s < _L0:
        sm = jnp.maximum(sm, _shl(sm, s, _PAD))
        s *= 2
    rmv = jnp.where(valid, jnp.minimum(cb, sm), jnp.float32(_PAD))

    byl = jnp.maximum(cb, ci)
    c1l = jnp.log(jnp.maximum(
        1.0 - jnp.exp(jnp.minimum(_shl(cb, 1, 0.0), _shl(ci, 2, 0.0))),
        1e-06))
    nnl = jnp.maximum(_shl(neg, 1, 0.0), _shl(neg, 2, 0.0))

    vals_ks = []
    rhs_ks = []
    im_hist = []
    jm_hist = []
    for k in range(_K):
        m6 = jnp.max(rmv, axis=1, keepdims=True)
        im6 = jnp.min(jnp.where(rmv == m6, jj, jnp.int32(_IBIG)),
                      axis=1, keepdims=True)
        bsel = jnp.min(jnp.where(jj == im6, cb, jnp.float32(_FBIG)),
                       axis=1, keepdims=True)
        row = jnp.minimum(bsel, xl) + jnp.where(
            jj < im6, jnp.float32(-1e8), jnp.float32(0.0))
        for t in range(k):
            row = jnp.where((im6 == im_hist[t]) & (jj == jm_hist[t]),
                            jnp.float32(_EXCL), row)
        jm6 = jnp.min(jnp.where(row == m6, jj, jnp.int32(_IBIG)),
                      axis=1, keepdims=True)
        rowx = jnp.where(jj == jm6, jnp.float32(_EXCL), row)
        rmv = jnp.where(jj == im6,
                        jnp.max(rowx, axis=1, keepdims=True), rmv)
        by_s = jnp.min(jnp.where(jj == im6, byl, jnp.float32(_FBIG)),
                       axis=1, keepdims=True)
        c1_s = jnp.min(jnp.where(jj == jm6, c1l, jnp.float32(_FBIG)),
                       axis=1, keepdims=True)
        nn_s = jnp.min(jnp.where(jj == jm6, nnl, jnp.float32(_FBIG)),
                       axis=1, keepdims=True)
        vals_ks.append(m6)
        rhs_ks.append(jnp.minimum(c1_s, jnp.maximum(by_s, nn_s)))
        im_hist.append(im6)
        jm_hist.append(jm6)

    loss = jnp.zeros((1, 1), jnp.float32)
    for k in range(_K):
        r6 = rhs_ks[k]
        m1 = jnp.min(r6, axis=0, keepdims=True)
        am = jnp.min(jnp.where(r6 == m1, ii6, jnp.int32(_IBIG)),
                     axis=0, keepdims=True)
        m2 = jnp.min(jnp.where(ii6 == am, jnp.float32(_FBIG), r6),
                     axis=0, keepdims=True)
        mex = jnp.where(ii6 == am, m2, m1)
        loss = loss + jnp.sum(jnp.maximum(vals_ks[k] - mex, 0.0),
                              axis=0, keepdims=True)
    return loss


def _body(ridx_ref, vl_ref, lp_ref, out_ref, gbuf_ref, sems_ref):
    copies = []
    for e in range(_B):
        copies.append(pltpu.make_async_copy(
            lp_ref.at[ridx_ref[e]], gbuf_ref.at[e], sems_ref.at[e]))
    for c in copies:
        c.start()

    jj = jax.lax.broadcasted_iota(jnp.int32, (_C, _L0), 1)
    valid = jj < _L
    ii6 = jax.lax.broadcasted_iota(jnp.int32, (_C, 1), 0)

    total = jnp.zeros((1, 1), jnp.float32)
    for ex in range(_B):
        copies[ex].wait()
        gt = jnp.transpose(gbuf_ref[ex], (1, 0))
        loss = _example_loss(gt, jj, valid, ii6)
        total = total + jnp.where(vl_ref[ex] > 0, loss,
                                  jnp.zeros((1, 1), jnp.float32))

    num_prop = vl_ref[0]
    for t in range(1, _B):
        num_prop = num_prop + vl_ref[t]
    out_ref[...] = total / jnp.maximum(num_prop, 1).astype(jnp.float32)


def kernel(log_pa, score, v_label, v_l, role_label, roleset_id, extra):
    b = log_pa.shape[0]
    v_idx = v_label[:, 0].astype(jnp.int32)
    row_idx = jnp.arange(b, dtype=jnp.int32) * _L0 + v_idx
    table = log_pa.reshape(b * _L0, _L0, _NL)

    out = pl.pallas_call(
        _body,
        grid_spec=pltpu.PrefetchScalarGridSpec(
            num_scalar_prefetch=2,
            grid=(1,),
            in_specs=[
                pl.BlockSpec(memory_space=pl.ANY),
            ],
            out_specs=pl.BlockSpec((1, 1), lambda i, v, vl: (0, 0)),
            scratch_shapes=[
                pltpu.VMEM((b, _L0, _NL), jnp.float32),
                pltpu.SemaphoreType.DMA((b,)),
            ],
        ),
        out_shape=jax.ShapeDtypeStruct((1, 1), jnp.float32),
    )(row_idx, v_l.astype(jnp.int32), table)
    return out.reshape(1)
